# baseline (speedup 1.0000x reference)
import functools

import jax
import jax.numpy as jnp
from jax import lax
from jax.experimental import pallas as pl
from jax.experimental.pallas import tpu as pltpu
from jax.experimental.pallas import tpu_sc as plsc

_NBUF = 2


def kernel(inputs, token_table, pos_table):
    B, L = inputs.shape
    V, D = token_table.shape

    info = plsc.get_sparse_core_info()
    NC, NS = info.num_cores, info.num_subcores
    NW = NC * NS

    n_seq = B // NW
    assert B % NW == 0 and n_seq % _NBUF == 0
    NV = D // 16
    CA = 128
    CB = L - CA
    HL = L // 2

    tab = jnp.pad(token_table, ((0, 0), (0, 128 - D)))
    idx = jnp.pad(inputs.astype(jnp.int32), ((0, 0), (0, 2 * CA - L))).reshape(
        B, 2, CA
    )
    posp = jnp.concatenate([pos_table[:HL], pos_table[HL:]], axis=1)

    mesh = plsc.VectorSubcoreMesh(core_axis_name="c", subcore_axis_name="s")

    @functools.partial(
        pl.kernel,
        out_type=jax.ShapeDtypeStruct((B, L, D), jnp.float32),
        mesh=mesh,
        compiler_params=pltpu.CompilerParams(use_tc_tiling_on_sc=True),
        scratch_types=[
            pltpu.VMEM((HL, 128), jnp.float32),
        ]
        + [pltpu.VMEM((2, CA), jnp.int32)] * _NBUF
        + [pltpu.VMEM((L, 128), jnp.float32)] * _NBUF
        + [pltpu.VMEM((L, D), jnp.float32)] * _NBUF
        + [pltpu.SemaphoreType.DMA] * (4 * _NBUF),
    )
    def emb_kernel(idx_hbm, tab_hbm, pos_hbm, out_hbm, pos_v, *rest):
        idxs = rest[:_NBUF]
        gbufs = rest[_NBUF : 2 * _NBUF]
        obufs = rest[2 * _NBUF : 3 * _NBUF]
        isems = rest[3 * _NBUF : 4 * _NBUF]
        asems = rest[4 * _NBUF : 5 * _NBUF]
        bsems = rest[5 * _NBUF : 6 * _NBUF]
        osems = rest[6 * _NBUF : 7 * _NBUF]

        wid = lax.axis_index("s") * NC + lax.axis_index("c")
        sbase = wid * n_seq
        pltpu.sync_copy(pos_hbm, pos_v)

        def start_idx(g, b):
            pltpu.async_copy(idx_hbm.at[sbase + g], idxs[b], isems[b])

        def wait_idx(g, b):
            pltpu.make_async_copy(idx_hbm.at[sbase + g], idxs[b], isems[b]).wait()

        def start_gather(b):
            pltpu.async_copy(tab_hbm.at[idxs[b].at[0]], gbufs[b].at[pl.ds(0, CA)], asems[b])
            pltpu.async_copy(
                tab_hbm.at[idxs[b].at[1].at[pl.ds(0, CB)]],
                gbufs[b].at[pl.ds(CA, CB)],
                bsems[b],
            )

        def wait_gather(b):
            pltpu.make_async_copy(
                tab_hbm.at[idxs[b].at[0]], gbufs[b].at[pl.ds(0, CA)], asems[b]
            ).wait()
            pltpu.make_async_copy(
                tab_hbm.at[idxs[b].at[1].at[pl.ds(0, CB)]],
                gbufs[b].at[pl.ds(CA, CB)],
                bsems[b],
            ).wait()

        def start_out(g, b):
            pltpu.async_copy(obufs[b], out_hbm.at[sbase + g], osems[b])

        def wait_out(g, b):
            pltpu.make_async_copy(obufs[b], out_hbm.at[sbase + g], osems[b]).wait()

        start_idx(0, 0)
        wait_idx(0, 0)
        start_gather(0)
        start_idx(1, 1)

        @pl.loop(0, n_seq, step=_NBUF)
        def _outer(g0):
            for b in range(_NBUF):
                g = g0 + b
                bn = (b + 1) % _NBUF

                @pl.when(g + 1 < n_seq)
                def _():
                    wait_idx(g + 1, bn)

                    @pl.when(g + 1 >= _NBUF)
                    def _():
                        wait_out(g + 1 - _NBUF, bn)

                    start_gather(bn)

                wait_gather(b)

                @pl.loop(0, L, unroll=2)
                def _row(r):
                    pr = lax.rem(r, HL)
                    ph = (r // HL) * D
                    for c in range(NV):
                        obufs[b][r, pl.ds(c * 16, 16)] = (
                            gbufs[b][r, pl.ds(c * 16, 16)]
                            + pos_v[pr, pl.ds(ph + c * 16, 16)]
                        )

                start_out(g, b)

                @pl.when(g + 2 < n_seq)
                def _():
                    start_idx(g + 2, b)

        for i in range(_NBUF):
            g = n_seq - _NBUF + i
            wait_out(g, g % _NBUF)

    return emb_kernel(idx, tab, posp)


# speedup vs baseline: 1.0618x; 1.0618x over previous
import functools

import jax
import jax.numpy as jnp
from jax import lax
from jax.experimental import pallas as pl
from jax.experimental.pallas import tpu as pltpu
from jax.experimental.pallas import tpu_sc as plsc

_NBUF = 8
_LEAD = 4
_NSLICE = 2


def _make_emb_kernel(Bs, L, D, NC, NS):
    NW = NC * NS
    R = Bs * L
    rows_per_w = R // NW
    CH = 100
    n_chunks = rows_per_w // CH
    assert rows_per_w % L == 0 and L % CH == 0 and rows_per_w % CH == 0
    assert n_chunks % _NBUF == 0
    SPC = L // CH
    NV = D // 16

    mesh = plsc.VectorSubcoreMesh(core_axis_name="c", subcore_axis_name="s")

    @functools.partial(
        pl.kernel,
        out_type=jax.ShapeDtypeStruct((Bs, L, D), jnp.float32),
        mesh=mesh,
        compiler_params=pltpu.CompilerParams(use_tc_tiling_on_sc=False),
        scratch_types=[
            pltpu.VMEM((n_chunks, CH), jnp.int32),
            pltpu.VMEM((L, D), jnp.float32),
        ]
        + [pltpu.VMEM((CH, D), jnp.float32)] * _NBUF
        + [pltpu.SemaphoreType.DMA] * (2 * _NBUF),
    )
    def emb_kernel(idx_hbm, tab_hbm, pos_hbm, out_hbm, idx_v, pos_v, *rest):
        bufs = rest[:_NBUF]
        gsems = rest[_NBUF : 2 * _NBUF]
        osems = rest[2 * _NBUF : 3 * _NBUF]

        wid = lax.axis_index("s") * NC + lax.axis_index("c")
        cbase = wid * n_chunks
        pltpu.sync_copy(idx_hbm.at[wid], idx_v)
        pltpu.sync_copy(pos_hbm, pos_v)

        def start_gather(g, b):
            pltpu.async_copy(tab_hbm.at[idx_v.at[g]], bufs[b], gsems[b])

        def wait_gather(g, b):
            pltpu.make_async_copy(tab_hbm.at[idx_v.at[g]], bufs[b], gsems[b]).wait()

        def out_slice(g):
            gg = cbase + g
            return out_hbm.at[gg // SPC, pl.ds(lax.rem(gg, SPC) * CH, CH)]

        def start_out(g, b):
            pltpu.async_copy(bufs[b], out_slice(g), osems[b])

        def wait_out(g, b):
            pltpu.make_async_copy(bufs[b], out_slice(g), osems[b]).wait()

        for b in range(_LEAD):
            start_gather(b, b)

        @pl.loop(0, n_chunks, step=_NBUF)
        def _outer(g0):
            for b in range(_NBUF):
                g = g0 + b
                bn = (b + _LEAD) % _NBUF
                if b < _LEAD:

                    @pl.when(g0 > 0)
                    def _():
                        wait_out(g - _LEAD, bn)

                    start_gather(g + _LEAD, bn)
                else:
                    wait_out(g - _LEAD, bn)

                    @pl.when(g0 < n_chunks - _NBUF)
                    def _():
                        start_gather(g + _LEAD, bn)

                wait_gather(g, b)
                pbase = lax.rem(g, SPC) * CH

                @pl.loop(0, CH, unroll=2)
                def _row(r):
                    for c in range(NV):
                        sl = pl.ds(c * 16, 16)
                        plsc.addupdate(bufs[b].at[r, sl], pos_v[pbase + r, sl])

                start_out(g, b)

        for i in range(_LEAD):
            g = n_chunks - _LEAD + i
            wait_out(g, g % _NBUF)

    return emb_kernel, n_chunks, CH, NW


def kernel(inputs, token_table, pos_table):
    B, L = inputs.shape
    V, D = token_table.shape

    info = plsc.get_sparse_core_info()
    NC, NS = info.num_cores, info.num_subcores
    NW = NC * NS

    Bs = B // _NSLICE
    emb_kernel, n_chunks, CH, NW = _make_emb_kernel(Bs, L, D, NC, NS)

    halves = []
    for s in range(_NSLICE):
        sl = inputs[s * Bs : (s + 1) * Bs]
        idx = sl.reshape(NW, n_chunks, CH).astype(jnp.int32)
        halves.append(emb_kernel(idx, token_table, pos_table))
    return jnp.concatenate(halves, axis=0)
